# baseline (speedup 1.0000x reference)
import functools

import numpy as np

import jax
import jax.numpy as jnp
from jax import lax
from jax.experimental import pallas as pl
from jax.experimental.pallas import tpu as pltpu
from jax.experimental.pallas import tpu_sc as plsc

N = 10000
D = 128
FH1 = 128
FH2 = 64
GH1 = 64
GH2 = 32
KC = 20
E = 320000
BN_EPS = 1e-4

NC = 2
NS = 16
NW = NC * NS
CHUNK = 128
EPW_CHUNKS = ((E + NW * CHUNK - 1) // (NW * CHUNK) + 7) // 8 * 8
E_PAD = NW * EPW_CHUNKS * CHUNK
ROWS_PER_TILE = (N // NS) // 8 * 8
ROWS_REM = N - NS * ROWS_PER_TILE

ROW_BLK = 1000
GRID = N // ROW_BLK


def _lane_bcast(v16, lane):
    idx = jnp.full((16, 1), lane, jnp.int32)
    return lax.gather(
        v16, idx,
        dimension_numbers=lax.GatherDimensionNumbers(
            offset_dims=(), collapsed_slice_dims=(0,), start_index_map=(0,)),
        slice_sizes=(1,),
        mode=lax.GatherScatterMode.PROMISE_IN_BOUNDS)


def _elu(h):
    return jnp.where(h > 0.0, h, jnp.exp(jnp.minimum(h, 0.0)) - 1.0)


NBUF = 6
PDIST = 4

FAST_C = 0
CF = 128
CS = 2 * EPW_CHUNKS - CF

_CP = np.zeros(FH2, np.int32)
for _h in (0, 1):
    for _j in range(16):
        _CP[32 * _h + 2 * _j] = 32 * _h + _j
        _CP[32 * _h + 2 * _j + 1] = 32 * _h + 16 + _j
COLPERM = _CP


NSBUF = 2


def _spmm_body(y_hbm, rows_hbm, cols_hbm, vals_hbm, out_hbm,
               rowv, colv, vb, gbuf, sbuf, acc, vsem, gsem, ssem):
    c = lax.axis_index("c")
    s = lax.axis_index("s")

    zv = jnp.zeros((16,), jnp.float32)
    for i in range(CHUNK):
        for f in range(FH2 // 16):
            sbuf[0, i, pl.ds(16 * f, 16)] = zv
    for j in range(ROWS_PER_TILE // CHUNK):
        pltpu.sync_copy(sbuf.at[0],
                        acc.at[pl.ds(s * ROWS_PER_TILE + j * CHUNK, CHUNK)])
    rem0 = ROWS_PER_TILE % CHUNK
    pltpu.sync_copy(
        sbuf.at[0, pl.ds(0, rem0)],
        acc.at[pl.ds(s * ROWS_PER_TILE + (ROWS_PER_TILE // CHUNK) * CHUNK,
                     rem0)])

    @pl.when(s == 0)
    def _zero_rem():
        pltpu.sync_copy(sbuf.at[0, pl.ds(0, ROWS_REM)],
                        acc.at[pl.ds(NS * ROWS_PER_TILE, ROWS_REM)])

    nch = jnp.where(c == FAST_C, CF, CS)
    base = s * (CF + CS) + jnp.where(c == FAST_C, 0, CF)

    @pl.when(c == FAST_C)
    def _stage_fast():
        pltpu.sync_copy(rows_hbm.at[pl.ds(base, CF)], rowv)
        pltpu.sync_copy(cols_hbm.at[pl.ds(base, CF)], colv)

    @pl.when(c != FAST_C)
    def _stage_slow():
        pltpu.sync_copy(rows_hbm.at[pl.ds(base, CS)], rowv.at[pl.ds(0, CS)])
        pltpu.sync_copy(cols_hbm.at[pl.ds(base, CS)], colv.at[pl.ds(0, CS)])

    plsc.subcore_barrier()

    def issue(tn, bp):
        pltpu.async_copy(vals_hbm.at[pl.ds((base + tn) * CHUNK, CHUNK)],
                         vb.at[bp], vsem.at[bp])
        pltpu.async_copy(y_hbm.at[colv.at[tn]], gbuf.at[bp], gsem.at[bp])

    for tb in range(PDIST):
        issue(tb, tb)

    def slot(t, carry):
        b = lax.rem(t, NBUF)
        bp = lax.rem(t + PDIST, NBUF)
        sb = lax.rem(t, NSBUF)

        @pl.when(t >= NSBUF)
        def _drain_prev():
            pltpu.make_async_copy(sbuf.at[sb], acc.at[rowv.at[t - NSBUF]],
                                  ssem.at[sb]).wait()

        @pl.when(t + PDIST < nch)
        def _prefetch():
            issue(t + PDIST, bp)

        pltpu.make_async_copy(vals_hbm.at[pl.ds((base + t) * CHUNK, CHUNK)],
                              vb.at[b], vsem.at[b]).wait()
        pltpu.make_async_copy(y_hbm.at[colv.at[t]], gbuf.at[b],
                              gsem.at[b]).wait()

        for g in range(CHUNK // 16):
            v16 = vb[b, pl.ds(16 * g, 16)]
            for e16 in range(16):
                e = 16 * g + e16
                vv = _lane_bcast(v16, e16)
                for h in range(2):
                    wb = gbuf[b, e, pl.ds(32 * h, 32)]
                    lo, hi = plsc.unpack(wb, format=plsc.PackFormat.INTERLEAVED)
                    sbuf[sb, e, pl.ds(32 * h, 16)] = lo * vv
                    sbuf[sb, e, pl.ds(32 * h + 16, 16)] = hi * vv

        pltpu.async_copy(sbuf.at[sb], acc.at[rowv.at[t]], ssem.at[sb],
                         add=True)
        return carry

    lax.fori_loop(0, nch, slot, 0)

    for back in range(NSBUF, 0, -1):
        tl = nch - back
        bl = lax.rem(tl, NSBUF)
        pltpu.make_async_copy(sbuf.at[bl], acc.at[rowv.at[tl]],
                              ssem.at[bl]).wait()
    plsc.subcore_barrier()

    pltpu.sync_copy(acc.at[pl.ds(s * ROWS_PER_TILE, ROWS_PER_TILE)],
                    out_hbm.at[c, pl.ds(s * ROWS_PER_TILE, ROWS_PER_TILE)])

    @pl.when(s == 0)
    def _write_rem():
        pltpu.sync_copy(acc.at[pl.ds(NS * ROWS_PER_TILE, ROWS_REM)],
                        out_hbm.at[c, pl.ds(NS * ROWS_PER_TILE, ROWS_REM)])


def _make_spmm():
    mesh = plsc.VectorSubcoreMesh(core_axis_name="c", subcore_axis_name="s")
    return pl.kernel(
        _spmm_body,
        out_type=jax.ShapeDtypeStruct((NC, N, FH2), jnp.float32),
        mesh=mesh,
        scratch_types=[
            pltpu.VMEM((CF, CHUNK), jnp.int32),
            pltpu.VMEM((CF, CHUNK), jnp.int32),
            pltpu.VMEM((NBUF, CHUNK), jnp.float32),
            pltpu.VMEM((NBUF, CHUNK, FH2), jnp.bfloat16),
            pltpu.VMEM((NSBUF, CHUNK, FH2), jnp.float32),
            pltpu.VMEM_SHARED((N, FH2), jnp.float32),
            pltpu.SemaphoreType.DMA((NBUF,)),
            pltpu.SemaphoreType.DMA((NBUF,)),
            pltpu.SemaphoreType.DMA((NSBUF,)),
        ],
        compiler_params=pltpu.CompilerParams(use_tc_tiling_on_sc=False,
                                             needs_layout_passes=False),
    )


def _enc_body(x_ref, w1_ref, b1_ref, g1_ref, be1_ref,
              w2_ref, b2_ref, g2_ref, be2_ref, gc1_ref,
              feat_ref, y1_ref):
    h = jnp.dot(x_ref[...], w1_ref[...], preferred_element_type=jnp.float32)
    h = (h + b1_ref[...]) * g1_ref[...] + be1_ref[...]
    h = _elu(h)
    f2 = jnp.dot(h, w2_ref[...], preferred_element_type=jnp.float32)
    f2 = (f2 + b2_ref[...]) * g2_ref[...] + be2_ref[...]
    f2 = _elu(f2)
    feat_ref[...] = f2
    y1_ref[...] = jnp.dot(
        f2, gc1_ref[...], preferred_element_type=jnp.float32
    ).astype(jnp.bfloat16)


def _mid_body(s1a_ref, s1b_ref, m2_ref, gc23_ref, y23_ref):
    h = jnp.maximum(s1a_ref[...] + s1b_ref[...], 0.0) * m2_ref[...]
    y23_ref[...] = jnp.dot(
        h, gc23_ref[...], preferred_element_type=jnp.float32
    ).astype(jnp.bfloat16)


def _tail_body(s2a_ref, s2b_ref, feat_ref, decw_ref, decb_ref, decg_ref,
               decbe_ref, ct_ref, c2_ref,
               s23_ref, z_ref, de_ref, q_ref):
    s23 = s2a_ref[...] + s2b_ref[...]
    s23_ref[...] = s23
    mu = s23[:, :GH2]
    z = jnp.concatenate([feat_ref[...], mu], axis=1)
    z_ref[...] = z
    de = jnp.dot(z, decw_ref[...], preferred_element_type=jnp.float32)
    de = (de + decb_ref[...]) * decg_ref[...] + decbe_ref[...]
    de_ref[...] = _elu(de)
    zc = jnp.dot(z, ct_ref[...], preferred_element_type=jnp.float32)
    z2 = jnp.sum(z * z, axis=1, keepdims=True)
    d2 = z2 - 2.0 * zc + c2_ref[...]
    col = lax.broadcasted_iota(jnp.int32, (ROW_BLK, 128), 1)
    qun = jnp.where(col < KC, 1.0 / (1.0 + d2), 0.0)
    q_ref[...] = qun / jnp.sum(qun, axis=1, keepdims=True)


def _row_spec(width):
    return pl.BlockSpec((ROW_BLK, width), lambda i: (i, 0))


def _full_spec(r, c):
    return pl.BlockSpec((r, c), lambda i: (0, 0))


def kernel(x, adj_indices, adj_values, training,
           enc1_W, enc1_b, enc1_gamma, enc1_beta,
           enc2_W, enc2_b, enc2_gamma, enc2_beta,
           gc1_W, gc2_W, gc3_W,
           dec_W, dec_b, dec_gamma, dec_beta, cluster):
    f32 = jnp.float32
    rs = 1.0 / jnp.sqrt(jnp.asarray(1.0 + BN_EPS, f32))

    ai = adj_indices.astype(jnp.int32)
    pad = E_PAD - E
    rows2d = jnp.pad(ai[0], (0, pad)).reshape(NW * EPW_CHUNKS, CHUNK)
    cols2d = jnp.pad(ai[1], (0, pad)).reshape(NW * EPW_CHUNKS, CHUNK)
    vals1d = jnp.pad(adj_values, (0, pad))

    b1 = enc1_b.reshape(1, FH1)
    g1 = (enc1_gamma * rs).reshape(1, FH1)
    be1 = enc1_beta.reshape(1, FH1)
    b2 = enc2_b.reshape(1, FH2)
    g2 = (enc2_gamma * rs).reshape(1, FH2)
    be2 = enc2_beta.reshape(1, FH2)
    gc1p = gc1_W[:, COLPERM]
    gc23p = jnp.concatenate([gc2_W, gc3_W], axis=1)[:, COLPERM]
    mask2 = jax.random.bernoulli(
        jax.random.key(42), 0.5, (N, GH1)).astype(f32) * 2.0
    decb = dec_b.reshape(1, D)
    decg = (dec_gamma * rs).reshape(1, D)
    decbe = dec_beta.reshape(1, D)
    ct = jnp.zeros((FH2 + GH2, 128), f32).at[:, :KC].set(cluster.T)
    c2 = jnp.zeros((1, 128), f32).at[:, :KC].set(
        jnp.sum(cluster * cluster, axis=1)[None, :])

    feat_x, y1 = pl.pallas_call(
        _enc_body,
        grid=(GRID,),
        in_specs=[
            _row_spec(D), _full_spec(D, FH1), _full_spec(1, FH1),
            _full_spec(1, FH1), _full_spec(1, FH1), _full_spec(FH1, FH2),
            _full_spec(1, FH2), _full_spec(1, FH2), _full_spec(1, FH2),
            _full_spec(FH2, GH1),
        ],
        out_specs=[_row_spec(FH2), _row_spec(GH1)],
        out_shape=[jax.ShapeDtypeStruct((N, FH2), f32),
                   jax.ShapeDtypeStruct((N, GH1), jnp.bfloat16)],
    )(x, enc1_W, b1, g1, be1, enc2_W, b2, g2, be2, gc1p)

    spmm = _make_spmm()

    p1 = spmm(y1, rows2d, cols2d, vals1d)

    y23 = pl.pallas_call(
        _mid_body,
        grid=(GRID,),
        in_specs=[_row_spec(GH1), _row_spec(GH1), _row_spec(GH1),
                  _full_spec(GH1, 2 * GH2)],
        out_specs=_row_spec(2 * GH2),
        out_shape=jax.ShapeDtypeStruct((N, 2 * GH2), jnp.bfloat16),
    )(p1[0], p1[1], mask2, gc23p)

    p2 = spmm(y23, rows2d, cols2d, vals1d)

    s23, z, de_feat, q_pad = pl.pallas_call(
        _tail_body,
        grid=(GRID,),
        in_specs=[_row_spec(2 * GH2), _row_spec(2 * GH2), _row_spec(FH2),
                  _full_spec(FH2 + GH2, D), _full_spec(1, D),
                  _full_spec(1, D), _full_spec(1, D),
                  _full_spec(FH2 + GH2, 128), _full_spec(1, 128)],
        out_specs=[_row_spec(2 * GH2), _row_spec(FH2 + GH2), _row_spec(D),
                   _row_spec(128)],
        out_shape=[jax.ShapeDtypeStruct((N, 2 * GH2), f32),
                   jax.ShapeDtypeStruct((N, FH2 + GH2), f32),
                   jax.ShapeDtypeStruct((N, D), f32),
                   jax.ShapeDtypeStruct((N, 128), f32)],
    )(p2[0], p2[1], feat_x, dec_W, decb, decg, decbe, ct, c2)

    mu = s23[:, :GH2]
    logvar = s23[:, GH2:]
    q = q_pad[:, :KC]
    return (z, mu, logvar, de_feat, q, feat_x, mu)


# speedup vs baseline: 1.1198x; 1.1198x over previous
import functools

import numpy as np

import jax
import jax.numpy as jnp
from jax import lax
from jax.experimental import pallas as pl
from jax.experimental.pallas import tpu as pltpu
from jax.experimental.pallas import tpu_sc as plsc

N = 10000
D = 128
FH1 = 128
FH2 = 64
GH1 = 64
GH2 = 32
KC = 20
E = 320000
BN_EPS = 1e-4

NC = 2
NS = 16
NW = NC * NS
CHUNK = 128
EPW_CHUNKS = ((E + NW * CHUNK - 1) // (NW * CHUNK) + 7) // 8 * 8
E_PAD = NW * EPW_CHUNKS * CHUNK
ROWS_PER_TILE = (N // NS) // 8 * 8
ROWS_REM = N - NS * ROWS_PER_TILE

ROW_BLK = 1000
GRID = N // ROW_BLK


def _lane_bcast(v16, lane):
    idx = jnp.full((16, 1), lane, jnp.int32)
    return lax.gather(
        v16, idx,
        dimension_numbers=lax.GatherDimensionNumbers(
            offset_dims=(), collapsed_slice_dims=(0,), start_index_map=(0,)),
        slice_sizes=(1,),
        mode=lax.GatherScatterMode.PROMISE_IN_BOUNDS)


def _elu(h):
    return jnp.where(h > 0.0, h, jnp.exp(jnp.minimum(h, 0.0)) - 1.0)


NBUF = 6
PDIST = 4

FAST_C = 0
CF = 104
CS = 2 * EPW_CHUNKS - CF

_CP = np.zeros(FH2, np.int32)
for _h in (0, 1):
    for _j in range(16):
        _CP[32 * _h + 2 * _j] = 32 * _h + _j
        _CP[32 * _h + 2 * _j + 1] = 32 * _h + 16 + _j
COLPERM = _CP


NSBUF = 2


def _spmm_body(y_hbm, rows_hbm, cols_hbm, vals_hbm, out_hbm,
               rowv, colv, vb, gbuf, sbuf, acc, vsem, gsem, ssem):
    c = lax.axis_index("c")
    s = lax.axis_index("s")

    zv = jnp.zeros((16,), jnp.float32)
    for i in range(CHUNK):
        for f in range(FH2 // 16):
            sbuf[0, i, pl.ds(16 * f, 16)] = zv
    for j in range(ROWS_PER_TILE // CHUNK):
        pltpu.sync_copy(sbuf.at[0],
                        acc.at[pl.ds(s * ROWS_PER_TILE + j * CHUNK, CHUNK)])
    rem0 = ROWS_PER_TILE % CHUNK
    pltpu.sync_copy(
        sbuf.at[0, pl.ds(0, rem0)],
        acc.at[pl.ds(s * ROWS_PER_TILE + (ROWS_PER_TILE // CHUNK) * CHUNK,
                     rem0)])

    @pl.when(s == 0)
    def _zero_rem():
        pltpu.sync_copy(sbuf.at[0, pl.ds(0, ROWS_REM)],
                        acc.at[pl.ds(NS * ROWS_PER_TILE, ROWS_REM)])

    nch = jnp.where(c == FAST_C, CF, CS)
    base = s * (CF + CS) + jnp.where(c == FAST_C, 0, CF)

    @pl.when(c == FAST_C)
    def _stage_fast():
        pltpu.sync_copy(rows_hbm.at[pl.ds(base, CF)], rowv)
        pltpu.sync_copy(cols_hbm.at[pl.ds(base, CF)], colv)

    @pl.when(c != FAST_C)
    def _stage_slow():
        pltpu.sync_copy(rows_hbm.at[pl.ds(base, CS)], rowv.at[pl.ds(0, CS)])
        pltpu.sync_copy(cols_hbm.at[pl.ds(base, CS)], colv.at[pl.ds(0, CS)])

    plsc.subcore_barrier()

    def issue(tn, bp):
        pltpu.async_copy(vals_hbm.at[pl.ds((base + tn) * CHUNK, CHUNK)],
                         vb.at[bp], vsem.at[bp])
        pltpu.async_copy(y_hbm.at[colv.at[tn]], gbuf.at[bp], gsem.at[bp])

    for tb in range(PDIST):
        issue(tb, tb)

    def slot(t, carry):
        b = lax.rem(t, NBUF)
        bp = lax.rem(t + PDIST, NBUF)
        sb = lax.rem(t, NSBUF)

        @pl.when(t >= NSBUF)
        def _drain_prev():
            pltpu.make_async_copy(sbuf.at[sb], acc.at[rowv.at[t - NSBUF]],
                                  ssem.at[sb]).wait()

        @pl.when(t + PDIST < nch)
        def _prefetch():
            issue(t + PDIST, bp)

        pltpu.make_async_copy(vals_hbm.at[pl.ds((base + t) * CHUNK, CHUNK)],
                              vb.at[b], vsem.at[b]).wait()
        pltpu.make_async_copy(y_hbm.at[colv.at[t]], gbuf.at[b],
                              gsem.at[b]).wait()

        for g in range(CHUNK // 16):
            v16 = vb[b, pl.ds(16 * g, 16)]
            for e16 in range(16):
                e = 16 * g + e16
                vv = _lane_bcast(v16, e16)
                for h in range(2):
                    wi = gbuf[b, e, pl.ds(16 * h, 16)]
                    lo = plsc.bitcast(lax.shift_left(wi, 16), jnp.float32)
                    hi = plsc.bitcast(
                        jnp.bitwise_and(wi, jnp.int32(-65536)), jnp.float32)
                    sbuf[sb, e, pl.ds(32 * h, 16)] = lo * vv
                    sbuf[sb, e, pl.ds(32 * h + 16, 16)] = hi * vv

        pltpu.async_copy(sbuf.at[sb], acc.at[rowv.at[t]], ssem.at[sb],
                         add=True)
        return carry

    lax.fori_loop(0, nch, slot, 0)

    for back in range(NSBUF, 0, -1):
        tl = nch - back
        bl = lax.rem(tl, NSBUF)
        pltpu.make_async_copy(sbuf.at[bl], acc.at[rowv.at[tl]],
                              ssem.at[bl]).wait()
    plsc.subcore_barrier()

    pltpu.sync_copy(acc.at[pl.ds(s * ROWS_PER_TILE, ROWS_PER_TILE)],
                    out_hbm.at[c, pl.ds(s * ROWS_PER_TILE, ROWS_PER_TILE)])

    @pl.when(s == 0)
    def _write_rem():
        pltpu.sync_copy(acc.at[pl.ds(NS * ROWS_PER_TILE, ROWS_REM)],
                        out_hbm.at[c, pl.ds(NS * ROWS_PER_TILE, ROWS_REM)])


def _make_spmm():
    mesh = plsc.VectorSubcoreMesh(core_axis_name="c", subcore_axis_name="s")
    return pl.kernel(
        _spmm_body,
        out_type=jax.ShapeDtypeStruct((NC, N, FH2), jnp.float32),
        mesh=mesh,
        scratch_types=[
            pltpu.VMEM((CF, CHUNK), jnp.int32),
            pltpu.VMEM((CF, CHUNK), jnp.int32),
            pltpu.VMEM((NBUF, CHUNK), jnp.float32),
            pltpu.VMEM((NBUF, CHUNK, FH2 // 2), jnp.int32),
            pltpu.VMEM((NSBUF, CHUNK, FH2), jnp.float32),
            pltpu.VMEM_SHARED((N, FH2), jnp.float32),
            pltpu.SemaphoreType.DMA((NBUF,)),
            pltpu.SemaphoreType.DMA((NBUF,)),
            pltpu.SemaphoreType.DMA((NSBUF,)),
        ],
        compiler_params=pltpu.CompilerParams(use_tc_tiling_on_sc=False,
                                             needs_layout_passes=False),
    )


def _enc_body(x_ref, w1_ref, b1_ref, g1_ref, be1_ref,
              w2_ref, b2_ref, g2_ref, be2_ref, gc1_ref,
              feat_ref, y1_ref):
    h = jnp.dot(x_ref[...], w1_ref[...], preferred_element_type=jnp.float32)
    h = (h + b1_ref[...]) * g1_ref[...] + be1_ref[...]
    h = _elu(h)
    f2 = jnp.dot(h, w2_ref[...], preferred_element_type=jnp.float32)
    f2 = (f2 + b2_ref[...]) * g2_ref[...] + be2_ref[...]
    f2 = _elu(f2)
    feat_ref[...] = f2
    y1_ref[...] = jnp.dot(
        f2, gc1_ref[...], preferred_element_type=jnp.float32
    ).astype(jnp.bfloat16)


def _mid_body(s1a_ref, s1b_ref, m2_ref, gc23_ref, y23_ref):
    h = jnp.maximum(s1a_ref[...] + s1b_ref[...], 0.0) * m2_ref[...]
    y23_ref[...] = jnp.dot(
        h, gc23_ref[...], preferred_element_type=jnp.float32
    ).astype(jnp.bfloat16)


def _tail_body(s2a_ref, s2b_ref, feat_ref, decw_ref, decb_ref, decg_ref,
               decbe_ref, ct_ref, c2_ref,
               s23_ref, z_ref, de_ref, q_ref):
    s23 = s2a_ref[...] + s2b_ref[...]
    s23_ref[...] = s23
    mu = s23[:, :GH2]
    z = jnp.concatenate([feat_ref[...], mu], axis=1)
    z_ref[...] = z
    de = jnp.dot(z, decw_ref[...], preferred_element_type=jnp.float32)
    de = (de + decb_ref[...]) * decg_ref[...] + decbe_ref[...]
    de_ref[...] = _elu(de)
    zc = jnp.dot(z, ct_ref[...], preferred_element_type=jnp.float32)
    z2 = jnp.sum(z * z, axis=1, keepdims=True)
    d2 = z2 - 2.0 * zc + c2_ref[...]
    col = lax.broadcasted_iota(jnp.int32, (ROW_BLK, 128), 1)
    qun = jnp.where(col < KC, 1.0 / (1.0 + d2), 0.0)
    q_ref[...] = qun / jnp.sum(qun, axis=1, keepdims=True)


def _row_spec(width):
    return pl.BlockSpec((ROW_BLK, width), lambda i: (i, 0))


def _full_spec(r, c):
    return pl.BlockSpec((r, c), lambda i: (0, 0))


def kernel(x, adj_indices, adj_values, training,
           enc1_W, enc1_b, enc1_gamma, enc1_beta,
           enc2_W, enc2_b, enc2_gamma, enc2_beta,
           gc1_W, gc2_W, gc3_W,
           dec_W, dec_b, dec_gamma, dec_beta, cluster):
    f32 = jnp.float32
    rs = 1.0 / jnp.sqrt(jnp.asarray(1.0 + BN_EPS, f32))

    ai = adj_indices.astype(jnp.int32)
    pad = E_PAD - E
    rows2d = jnp.pad(ai[0], (0, pad)).reshape(NW * EPW_CHUNKS, CHUNK)
    cols2d = jnp.pad(ai[1], (0, pad)).reshape(NW * EPW_CHUNKS, CHUNK)
    vals1d = jnp.pad(adj_values, (0, pad))

    b1 = enc1_b.reshape(1, FH1)
    g1 = (enc1_gamma * rs).reshape(1, FH1)
    be1 = enc1_beta.reshape(1, FH1)
    b2 = enc2_b.reshape(1, FH2)
    g2 = (enc2_gamma * rs).reshape(1, FH2)
    be2 = enc2_beta.reshape(1, FH2)
    gc1p = gc1_W[:, COLPERM]
    gc23p = jnp.concatenate([gc2_W, gc3_W], axis=1)[:, COLPERM]
    mask2 = jax.random.bernoulli(
        jax.random.key(42), 0.5, (N, GH1)).astype(f32) * 2.0
    decb = dec_b.reshape(1, D)
    decg = (dec_gamma * rs).reshape(1, D)
    decbe = dec_beta.reshape(1, D)
    ct = jnp.zeros((FH2 + GH2, 128), f32).at[:, :KC].set(cluster.T)
    c2 = jnp.zeros((1, 128), f32).at[:, :KC].set(
        jnp.sum(cluster * cluster, axis=1)[None, :])

    feat_x, y1 = pl.pallas_call(
        _enc_body,
        grid=(GRID,),
        in_specs=[
            _row_spec(D), _full_spec(D, FH1), _full_spec(1, FH1),
            _full_spec(1, FH1), _full_spec(1, FH1), _full_spec(FH1, FH2),
            _full_spec(1, FH2), _full_spec(1, FH2), _full_spec(1, FH2),
            _full_spec(FH2, GH1),
        ],
        out_specs=[_row_spec(FH2), _row_spec(GH1)],
        out_shape=[jax.ShapeDtypeStruct((N, FH2), f32),
                   jax.ShapeDtypeStruct((N, GH1), jnp.bfloat16)],
    )(x, enc1_W, b1, g1, be1, enc2_W, b2, g2, be2, gc1p)

    spmm = _make_spmm()

    y1b = lax.bitcast_convert_type(y1.reshape(N, GH1 // 2, 2), jnp.int32)
    p1 = spmm(y1b, rows2d, cols2d, vals1d)

    y23 = pl.pallas_call(
        _mid_body,
        grid=(GRID,),
        in_specs=[_row_spec(GH1), _row_spec(GH1), _row_spec(GH1),
                  _full_spec(GH1, 2 * GH2)],
        out_specs=_row_spec(2 * GH2),
        out_shape=jax.ShapeDtypeStruct((N, 2 * GH2), jnp.bfloat16),
    )(p1[0], p1[1], mask2, gc23p)

    y23b = lax.bitcast_convert_type(y23.reshape(N, GH2, 2), jnp.int32)
    p2 = spmm(y23b, rows2d, cols2d, vals1d)

    s23, z, de_feat, q_pad = pl.pallas_call(
        _tail_body,
        grid=(GRID,),
        in_specs=[_row_spec(2 * GH2), _row_spec(2 * GH2), _row_spec(FH2),
                  _full_spec(FH2 + GH2, D), _full_spec(1, D),
                  _full_spec(1, D), _full_spec(1, D),
                  _full_spec(FH2 + GH2, 128), _full_spec(1, 128)],
        out_specs=[_row_spec(2 * GH2), _row_spec(FH2 + GH2), _row_spec(D),
                   _row_spec(128)],
        out_shape=[jax.ShapeDtypeStruct((N, 2 * GH2), f32),
                   jax.ShapeDtypeStruct((N, FH2 + GH2), f32),
                   jax.ShapeDtypeStruct((N, D), f32),
                   jax.ShapeDtypeStruct((N, 128), f32)],
    )(p2[0], p2[1], feat_x, dec_W, decb, decg, decbe, ct, c2)

    mu = s23[:, :GH2]
    logvar = s23[:, GH2:]
    q = q_pad[:, :KC]
    return (z, mu, logvar, de_feat, q, feat_x, mu)


# speedup vs baseline: 1.4320x; 1.2788x over previous
import functools

import numpy as np

import jax
import jax.numpy as jnp
from jax import lax
from jax.experimental import pallas as pl
from jax.experimental.pallas import tpu as pltpu
from jax.experimental.pallas import tpu_sc as plsc

N = 10000
D = 128
FH1 = 128
FH2 = 64
GH1 = 64
GH2 = 32
KC = 20
E = 320000
BN_EPS = 1e-4

NC = 2
NS = 16
NW = NC * NS
CHUNK = 128
EPW_CHUNKS = ((E + NW * CHUNK - 1) // (NW * CHUNK) + 7) // 8 * 8
E_PAD = NW * EPW_CHUNKS * CHUNK
ROWS_PER_TILE = (N // NS) // 8 * 8
ROWS_REM = N - NS * ROWS_PER_TILE

ROW_BLK = 1000
GRID = N // ROW_BLK


def _lane_bcast(v16, lane):
    idx = jnp.full((16, 1), lane, jnp.int32)
    return lax.gather(
        v16, idx,
        dimension_numbers=lax.GatherDimensionNumbers(
            offset_dims=(), collapsed_slice_dims=(0,), start_index_map=(0,)),
        slice_sizes=(1,),
        mode=lax.GatherScatterMode.PROMISE_IN_BOUNDS)


def _elu(h):
    return jnp.where(h > 0.0, h, jnp.exp(jnp.minimum(h, 0.0)) - 1.0)


NBUF = 6
PDIST = 4

FAST_C = 0
CF = 128
CS = 2 * EPW_CHUNKS - CF


def _spmm_body(y_hbm, rows_hbm, cols_hbm, vals_hbm, out_hbm,
               rowv, colv, vb, gbuf, acc, vsem, gsem, ssem):
    c = lax.axis_index("c")
    s = lax.axis_index("s")

    zv = jnp.zeros((16,), jnp.float32)
    for i in range(CHUNK):
        for f in range(FH2 // 16):
            gbuf[0, i, pl.ds(16 * f, 16)] = zv
    for j in range(ROWS_PER_TILE // CHUNK):
        pltpu.sync_copy(gbuf.at[0],
                        acc.at[pl.ds(s * ROWS_PER_TILE + j * CHUNK, CHUNK)])
    rem0 = ROWS_PER_TILE % CHUNK
    pltpu.sync_copy(
        gbuf.at[0, pl.ds(0, rem0)],
        acc.at[pl.ds(s * ROWS_PER_TILE + (ROWS_PER_TILE // CHUNK) * CHUNK,
                     rem0)])

    @pl.when(s == 0)
    def _zero_rem():
        pltpu.sync_copy(gbuf.at[0, pl.ds(0, ROWS_REM)],
                        acc.at[pl.ds(NS * ROWS_PER_TILE, ROWS_REM)])

    nch = jnp.where(c == FAST_C, CF, CS)
    base = s * (CF + CS) + jnp.where(c == FAST_C, 0, CF)

    @pl.when(c == FAST_C)
    def _stage_fast():
        pltpu.sync_copy(rows_hbm.at[pl.ds(base, CF)], rowv)
        pltpu.sync_copy(cols_hbm.at[pl.ds(base, CF)], colv)

    @pl.when(c != FAST_C)
    def _stage_slow():
        pltpu.sync_copy(rows_hbm.at[pl.ds(base, CS)], rowv.at[pl.ds(0, CS)])
        pltpu.sync_copy(cols_hbm.at[pl.ds(base, CS)], colv.at[pl.ds(0, CS)])

    plsc.subcore_barrier()

    def issue(tn, bp):
        pltpu.async_copy(vals_hbm.at[pl.ds((base + tn) * CHUNK, CHUNK)],
                         vb.at[bp], vsem.at[bp])
        pltpu.async_copy(y_hbm.at[colv.at[tn]], gbuf.at[bp], gsem.at[bp])

    for tb in range(PDIST):
        issue(tb, tb)

    def slot(t, carry):
        b = lax.rem(t, NBUF)
        bp = lax.rem(t + PDIST, NBUF)

        @pl.when(t >= NBUF - PDIST)
        def _drain_prev():
            pltpu.make_async_copy(gbuf.at[bp],
                                  acc.at[rowv.at[t - (NBUF - PDIST)]],
                                  ssem.at[bp]).wait()

        @pl.when(t + PDIST < nch)
        def _prefetch():
            issue(t + PDIST, bp)

        pltpu.make_async_copy(vals_hbm.at[pl.ds((base + t) * CHUNK, CHUNK)],
                              vb.at[b], vsem.at[b]).wait()
        pltpu.make_async_copy(y_hbm.at[colv.at[t]], gbuf.at[b],
                              gsem.at[b]).wait()

        for g in range(CHUNK // 16):
            v16 = vb[b, pl.ds(16 * g, 16)]
            for e16 in range(16):
                e = 16 * g + e16
                vv = _lane_bcast(v16, e16)
                for f in range(FH2 // 16):
                    sl = pl.ds(f * 16, 16)
                    gbuf[b, e, sl] = gbuf[b, e, sl] * vv

        pltpu.async_copy(gbuf.at[b], acc.at[rowv.at[t]], ssem.at[b],
                         add=True)
        return carry

    lax.fori_loop(0, nch, slot, 0)

    for back in range(NBUF - PDIST, 0, -1):
        tl = nch - back
        bl = lax.rem(tl, NBUF)
        pltpu.make_async_copy(gbuf.at[bl], acc.at[rowv.at[tl]],
                              ssem.at[bl]).wait()
    plsc.subcore_barrier()

    pltpu.sync_copy(acc.at[pl.ds(s * ROWS_PER_TILE, ROWS_PER_TILE)],
                    out_hbm.at[c, pl.ds(s * ROWS_PER_TILE, ROWS_PER_TILE)])

    @pl.when(s == 0)
    def _write_rem():
        pltpu.sync_copy(acc.at[pl.ds(NS * ROWS_PER_TILE, ROWS_REM)],
                        out_hbm.at[c, pl.ds(NS * ROWS_PER_TILE, ROWS_REM)])


def _make_spmm():
    mesh = plsc.VectorSubcoreMesh(core_axis_name="c", subcore_axis_name="s")
    return pl.kernel(
        _spmm_body,
        out_type=jax.ShapeDtypeStruct((NC, N, FH2), jnp.float32),
        mesh=mesh,
        scratch_types=[
            pltpu.VMEM((CF, CHUNK), jnp.int32),
            pltpu.VMEM((CF, CHUNK), jnp.int32),
            pltpu.VMEM((NBUF, CHUNK), jnp.float32),
            pltpu.VMEM((NBUF, CHUNK, FH2), jnp.float32),
            pltpu.VMEM_SHARED((N, FH2), jnp.float32),
            pltpu.SemaphoreType.DMA((NBUF,)),
            pltpu.SemaphoreType.DMA((NBUF,)),
            pltpu.SemaphoreType.DMA((NBUF,)),
        ],
        compiler_params=pltpu.CompilerParams(use_tc_tiling_on_sc=False),
    )


def _enc_body(x_ref, w1_ref, b1_ref, g1_ref, be1_ref,
              w2_ref, b2_ref, g2_ref, be2_ref, gc1_ref,
              feat_ref, y1_ref):
    h = jnp.dot(x_ref[...], w1_ref[...], preferred_element_type=jnp.float32)
    h = (h + b1_ref[...]) * g1_ref[...] + be1_ref[...]
    h = _elu(h)
    f2 = jnp.dot(h, w2_ref[...], preferred_element_type=jnp.float32)
    f2 = (f2 + b2_ref[...]) * g2_ref[...] + be2_ref[...]
    f2 = _elu(f2)
    feat_ref[...] = f2
    y1_ref[...] = jnp.dot(f2, gc1_ref[...], preferred_element_type=jnp.float32)


def _mid_body(s1a_ref, s1b_ref, m2_ref, gc23_ref, y23_ref):
    h = jnp.maximum(s1a_ref[...] + s1b_ref[...], 0.0) * m2_ref[...]
    y23_ref[...] = jnp.dot(h, gc23_ref[...],
                           preferred_element_type=jnp.float32)


def _tail_body(s2a_ref, s2b_ref, feat_ref, decw_ref, decb_ref, decg_ref,
               decbe_ref, ct_ref, c2_ref,
               s23_ref, z_ref, de_ref, q_ref):
    s23 = s2a_ref[...] + s2b_ref[...]
    s23_ref[...] = s23
    mu = s23[:, :GH2]
    z = jnp.concatenate([feat_ref[...], mu], axis=1)
    z_ref[...] = z
    de = jnp.dot(z, decw_ref[...], preferred_element_type=jnp.float32)
    de = (de + decb_ref[...]) * decg_ref[...] + decbe_ref[...]
    de_ref[...] = _elu(de)
    zc = jnp.dot(z, ct_ref[...], preferred_element_type=jnp.float32)
    z2 = jnp.sum(z * z, axis=1, keepdims=True)
    d2 = z2 - 2.0 * zc + c2_ref[...]
    col = lax.broadcasted_iota(jnp.int32, (ROW_BLK, 128), 1)
    qun = jnp.where(col < KC, 1.0 / (1.0 + d2), 0.0)
    q_ref[...] = qun / jnp.sum(qun, axis=1, keepdims=True)


def _row_spec(width):
    return pl.BlockSpec((ROW_BLK, width), lambda i: (i, 0))


def _full_spec(r, c):
    return pl.BlockSpec((r, c), lambda i: (0, 0))


def kernel(x, adj_indices, adj_values, training,
           enc1_W, enc1_b, enc1_gamma, enc1_beta,
           enc2_W, enc2_b, enc2_gamma, enc2_beta,
           gc1_W, gc2_W, gc3_W,
           dec_W, dec_b, dec_gamma, dec_beta, cluster):
    f32 = jnp.float32
    rs = 1.0 / jnp.sqrt(jnp.asarray(1.0 + BN_EPS, f32))

    ai = adj_indices.astype(jnp.int32)
    pad = E_PAD - E
    rows2d = jnp.pad(ai[0], (0, pad)).reshape(NW * EPW_CHUNKS, CHUNK)
    cols2d = jnp.pad(ai[1], (0, pad)).reshape(NW * EPW_CHUNKS, CHUNK)
    vals1d = jnp.pad(adj_values, (0, pad))

    b1 = enc1_b.reshape(1, FH1)
    g1 = (enc1_gamma * rs).reshape(1, FH1)
    be1 = enc1_beta.reshape(1, FH1)
    b2 = enc2_b.reshape(1, FH2)
    g2 = (enc2_gamma * rs).reshape(1, FH2)
    be2 = enc2_beta.reshape(1, FH2)
    gc23 = jnp.concatenate([gc2_W, gc3_W], axis=1)
    mask2 = jax.random.bernoulli(
        jax.random.key(42), 0.5, (N, GH1)).astype(f32) * 2.0
    decb = dec_b.reshape(1, D)
    decg = (dec_gamma * rs).reshape(1, D)
    decbe = dec_beta.reshape(1, D)
    ct = jnp.zeros((FH2 + GH2, 128), f32).at[:, :KC].set(cluster.T)
    c2 = jnp.zeros((1, 128), f32).at[:, :KC].set(
        jnp.sum(cluster * cluster, axis=1)[None, :])

    feat_x, y1 = pl.pallas_call(
        _enc_body,
        grid=(GRID,),
        in_specs=[
            _row_spec(D), _full_spec(D, FH1), _full_spec(1, FH1),
            _full_spec(1, FH1), _full_spec(1, FH1), _full_spec(FH1, FH2),
            _full_spec(1, FH2), _full_spec(1, FH2), _full_spec(1, FH2),
            _full_spec(FH2, GH1),
        ],
        out_specs=[_row_spec(FH2), _row_spec(GH1)],
        out_shape=[jax.ShapeDtypeStruct((N, FH2), f32),
                   jax.ShapeDtypeStruct((N, GH1), f32)],
    )(x, enc1_W, b1, g1, be1, enc2_W, b2, g2, be2, gc1_W)

    spmm = _make_spmm()

    p1 = spmm(y1, rows2d, cols2d, vals1d)

    y23 = pl.pallas_call(
        _mid_body,
        grid=(GRID,),
        in_specs=[_row_spec(GH1), _row_spec(GH1), _row_spec(GH1),
                  _full_spec(GH1, 2 * GH2)],
        out_specs=_row_spec(2 * GH2),
        out_shape=jax.ShapeDtypeStruct((N, 2 * GH2), f32),
    )(p1[0], p1[1], mask2, gc23)

    p2 = spmm(y23, rows2d, cols2d, vals1d)

    s23, z, de_feat, q_pad = pl.pallas_call(
        _tail_body,
        grid=(GRID,),
        in_specs=[_row_spec(2 * GH2), _row_spec(2 * GH2), _row_spec(FH2),
                  _full_spec(FH2 + GH2, D), _full_spec(1, D),
                  _full_spec(1, D), _full_spec(1, D),
                  _full_spec(FH2 + GH2, 128), _full_spec(1, 128)],
        out_specs=[_row_spec(2 * GH2), _row_spec(FH2 + GH2), _row_spec(D),
                   _row_spec(128)],
        out_shape=[jax.ShapeDtypeStruct((N, 2 * GH2), f32),
                   jax.ShapeDtypeStruct((N, FH2 + GH2), f32),
                   jax.ShapeDtypeStruct((N, D), f32),
                   jax.ShapeDtypeStruct((N, 128), f32)],
    )(p2[0], p2[1], feat_x, dec_W, decb, decg, decbe, ct, c2)

    mu = s23[:, :GH2]
    logvar = s23[:, GH2:]
    q = q_pad[:, :KC]
    return (z, mu, logvar, de_feat, q, feat_x, mu)


# speedup vs baseline: 1.4621x; 1.0210x over previous
import functools

import numpy as np

import jax
import jax.numpy as jnp
from jax import lax
from jax.experimental import pallas as pl
from jax.experimental.pallas import tpu as pltpu
from jax.experimental.pallas import tpu_sc as plsc

N = 10000
D = 128
FH1 = 128
FH2 = 64
GH1 = 64
GH2 = 32
KC = 20
E = 320000
BN_EPS = 1e-4

NC = 2
NS = 16
NW = NC * NS
CHUNK = 128
EPW_CHUNKS = ((E + NW * CHUNK - 1) // (NW * CHUNK) + 7) // 8 * 8
E_PAD = NW * EPW_CHUNKS * CHUNK
ROWS_PER_TILE = (N // NS) // 8 * 8
ROWS_REM = N - NS * ROWS_PER_TILE

ROW_BLK = 1000
GRID = N // ROW_BLK


def _lane_bcast(v16, lane):
    idx = jnp.full((16, 1), lane, jnp.int32)
    return lax.gather(
        v16, idx,
        dimension_numbers=lax.GatherDimensionNumbers(
            offset_dims=(), collapsed_slice_dims=(0,), start_index_map=(0,)),
        slice_sizes=(1,),
        mode=lax.GatherScatterMode.PROMISE_IN_BOUNDS)


def _elu(h):
    return jnp.where(h > 0.0, h, jnp.exp(jnp.minimum(h, 0.0)) - 1.0)


NBUF = 6
PDIST = 4

FAST_C = 0
CF = 120
CS = 2 * EPW_CHUNKS - CF


def _spmm_body(y_hbm, rows_hbm, cols_hbm, vals_hbm, out_hbm,
               rowv, colv, vb, gbuf, acc, vsem, gsem, ssem):
    c = lax.axis_index("c")
    s = lax.axis_index("s")

    zv = jnp.zeros((16,), jnp.float32)
    for i in range(CHUNK):
        for f in range(FH2 // 16):
            gbuf[0, i, pl.ds(16 * f, 16)] = zv
    for j in range(ROWS_PER_TILE // CHUNK):
        pltpu.sync_copy(gbuf.at[0],
                        acc.at[pl.ds(s * ROWS_PER_TILE + j * CHUNK, CHUNK)])
    rem0 = ROWS_PER_TILE % CHUNK
    pltpu.sync_copy(
        gbuf.at[0, pl.ds(0, rem0)],
        acc.at[pl.ds(s * ROWS_PER_TILE + (ROWS_PER_TILE // CHUNK) * CHUNK,
                     rem0)])

    @pl.when(s == 0)
    def _zero_rem():
        pltpu.sync_copy(gbuf.at[0, pl.ds(0, ROWS_REM)],
                        acc.at[pl.ds(NS * ROWS_PER_TILE, ROWS_REM)])

    nch = jnp.where(c == FAST_C, CF, CS)
    base = s * (CF + CS) + jnp.where(c == FAST_C, 0, CF)

    @pl.when(c == FAST_C)
    def _stage_fast():
        pltpu.sync_copy(rows_hbm.at[pl.ds(base, CF)], rowv)
        pltpu.sync_copy(cols_hbm.at[pl.ds(base, CF)], colv)

    @pl.when(c != FAST_C)
    def _stage_slow():
        pltpu.sync_copy(rows_hbm.at[pl.ds(base, CS)], rowv.at[pl.ds(0, CS)])
        pltpu.sync_copy(cols_hbm.at[pl.ds(base, CS)], colv.at[pl.ds(0, CS)])

    plsc.subcore_barrier()

    def issue(tn, bp):
        pltpu.async_copy(vals_hbm.at[pl.ds((base + tn) * CHUNK, CHUNK)],
                         vb.at[bp], vsem.at[bp])
        pltpu.async_copy(y_hbm.at[colv.at[tn]], gbuf.at[bp], gsem.at[bp])

    for tb in range(PDIST):
        issue(tb, tb)

    def slot(t, carry):
        b = lax.rem(t, NBUF)
        bp = lax.rem(t + PDIST, NBUF)

        @pl.when(t >= NBUF - PDIST)
        def _drain_prev():
            pltpu.make_async_copy(gbuf.at[bp],
                                  acc.at[rowv.at[t - (NBUF - PDIST)]],
                                  ssem.at[bp]).wait()

        @pl.when(t + PDIST < nch)
        def _prefetch():
            issue(t + PDIST, bp)

        pltpu.make_async_copy(vals_hbm.at[pl.ds((base + t) * CHUNK, CHUNK)],
                              vb.at[b], vsem.at[b]).wait()
        pltpu.make_async_copy(y_hbm.at[colv.at[t]], gbuf.at[b],
                              gsem.at[b]).wait()

        for g in range(CHUNK // 16):
            v16 = vb[b, pl.ds(16 * g, 16)]
            for e16 in range(16):
                e = 16 * g + e16
                vv = _lane_bcast(v16, e16)
                for f in range(FH2 // 16):
                    sl = pl.ds(f * 16, 16)
                    gbuf[b, e, sl] = gbuf[b, e, sl] * vv

        pltpu.async_copy(gbuf.at[b], acc.at[rowv.at[t]], ssem.at[b],
                         add=True)
        return carry

    lax.fori_loop(0, nch, slot, 0)

    for back in range(NBUF - PDIST, 0, -1):
        tl = nch - back
        bl = lax.rem(tl, NBUF)
        pltpu.make_async_copy(gbuf.at[bl], acc.at[rowv.at[tl]],
                              ssem.at[bl]).wait()
    plsc.subcore_barrier()

    pltpu.sync_copy(acc.at[pl.ds(s * ROWS_PER_TILE, ROWS_PER_TILE)],
                    out_hbm.at[c, pl.ds(s * ROWS_PER_TILE, ROWS_PER_TILE)])

    @pl.when(s == 0)
    def _write_rem():
        pltpu.sync_copy(acc.at[pl.ds(NS * ROWS_PER_TILE, ROWS_REM)],
                        out_hbm.at[c, pl.ds(NS * ROWS_PER_TILE, ROWS_REM)])


def _make_spmm():
    mesh = plsc.VectorSubcoreMesh(core_axis_name="c", subcore_axis_name="s")
    return pl.kernel(
        _spmm_body,
        out_type=jax.ShapeDtypeStruct((NC, N, FH2), jnp.float32),
        mesh=mesh,
        scratch_types=[
            pltpu.VMEM((CF, CHUNK), jnp.int32),
            pltpu.VMEM((CF, CHUNK), jnp.int32),
            pltpu.VMEM((NBUF, CHUNK), jnp.float32),
            pltpu.VMEM((NBUF, CHUNK, FH2), jnp.float32),
            pltpu.VMEM_SHARED((N, FH2), jnp.float32),
            pltpu.SemaphoreType.DMA((NBUF,)),
            pltpu.SemaphoreType.DMA((NBUF,)),
            pltpu.SemaphoreType.DMA((NBUF,)),
        ],
        compiler_params=pltpu.CompilerParams(use_tc_tiling_on_sc=False),
    )


def _enc_body(x_ref, w1_ref, b1_ref, g1_ref, be1_ref,
              w2_ref, b2_ref, g2_ref, be2_ref, gc1_ref,
              feat_ref, y1_ref):
    h = jnp.dot(x_ref[...], w1_ref[...], preferred_element_type=jnp.float32)
    h = (h + b1_ref[...]) * g1_ref[...] + be1_ref[...]
    h = _elu(h)
    f2 = jnp.dot(h, w2_ref[...], preferred_element_type=jnp.float32)
    f2 = (f2 + b2_ref[...]) * g2_ref[...] + be2_ref[...]
    f2 = _elu(f2)
    feat_ref[...] = f2
    y1_ref[...] = jnp.dot(f2, gc1_ref[...], preferred_element_type=jnp.float32)


def _mid_body(s1a_ref, s1b_ref, m2_ref, gc23_ref, y23_ref):
    h = jnp.maximum(s1a_ref[...] + s1b_ref[...], 0.0) * m2_ref[...]
    y23_ref[...] = jnp.dot(h, gc23_ref[...],
                           preferred_element_type=jnp.float32)


def _tail_body(s2a_ref, s2b_ref, feat_ref, decw_ref, decb_ref, decg_ref,
               decbe_ref, ct_ref, c2_ref,
               s23_ref, z_ref, de_ref, q_ref):
    s23 = s2a_ref[...] + s2b_ref[...]
    s23_ref[...] = s23
    mu = s23[:, :GH2]
    z = jnp.concatenate([feat_ref[...], mu], axis=1)
    z_ref[...] = z
    de = jnp.dot(z, decw_ref[...], preferred_element_type=jnp.float32)
    de = (de + decb_ref[...]) * decg_ref[...] + decbe_ref[...]
    de_ref[...] = _elu(de)
    zc = jnp.dot(z, ct_ref[...], preferred_element_type=jnp.float32)
    z2 = jnp.sum(z * z, axis=1, keepdims=True)
    d2 = z2 - 2.0 * zc + c2_ref[...]
    col = lax.broadcasted_iota(jnp.int32, (ROW_BLK, 128), 1)
    qun = jnp.where(col < KC, 1.0 / (1.0 + d2), 0.0)
    q_ref[...] = qun / jnp.sum(qun, axis=1, keepdims=True)


def _row_spec(width):
    return pl.BlockSpec((ROW_BLK, width), lambda i: (i, 0))


def _full_spec(r, c):
    return pl.BlockSpec((r, c), lambda i: (0, 0))


def kernel(x, adj_indices, adj_values, training,
           enc1_W, enc1_b, enc1_gamma, enc1_beta,
           enc2_W, enc2_b, enc2_gamma, enc2_beta,
           gc1_W, gc2_W, gc3_W,
           dec_W, dec_b, dec_gamma, dec_beta, cluster):
    f32 = jnp.float32
    rs = 1.0 / jnp.sqrt(jnp.asarray(1.0 + BN_EPS, f32))

    ai = adj_indices.astype(jnp.int32)
    pad = E_PAD - E
    rows2d = jnp.pad(ai[0], (0, pad)).reshape(NW * EPW_CHUNKS, CHUNK)
    cols2d = jnp.pad(ai[1], (0, pad)).reshape(NW * EPW_CHUNKS, CHUNK)
    vals1d = jnp.pad(adj_values, (0, pad))

    b1 = enc1_b.reshape(1, FH1)
    g1 = (enc1_gamma * rs).reshape(1, FH1)
    be1 = enc1_beta.reshape(1, FH1)
    b2 = enc2_b.reshape(1, FH2)
    g2 = (enc2_gamma * rs).reshape(1, FH2)
    be2 = enc2_beta.reshape(1, FH2)
    gc23 = jnp.concatenate([gc2_W, gc3_W], axis=1)
    mask2 = jax.random.bernoulli(
        jax.random.key(42), 0.5, (N, GH1)).astype(f32) * 2.0
    decb = dec_b.reshape(1, D)
    decg = (dec_gamma * rs).reshape(1, D)
    decbe = dec_beta.reshape(1, D)
    ct = jnp.zeros((FH2 + GH2, 128), f32).at[:, :KC].set(cluster.T)
    c2 = jnp.zeros((1, 128), f32).at[:, :KC].set(
        jnp.sum(cluster * cluster, axis=1)[None, :])

    feat_x, y1 = pl.pallas_call(
        _enc_body,
        grid=(GRID,),
        in_specs=[
            _row_spec(D), _full_spec(D, FH1), _full_spec(1, FH1),
            _full_spec(1, FH1), _full_spec(1, FH1), _full_spec(FH1, FH2),
            _full_spec(1, FH2), _full_spec(1, FH2), _full_spec(1, FH2),
            _full_spec(FH2, GH1),
        ],
        out_specs=[_row_spec(FH2), _row_spec(GH1)],
        out_shape=[jax.ShapeDtypeStruct((N, FH2), f32),
                   jax.ShapeDtypeStruct((N, GH1), f32)],
    )(x, enc1_W, b1, g1, be1, enc2_W, b2, g2, be2, gc1_W)

    spmm = _make_spmm()

    p1 = spmm(y1, rows2d, cols2d, vals1d)

    y23 = pl.pallas_call(
        _mid_body,
        grid=(GRID,),
        in_specs=[_row_spec(GH1), _row_spec(GH1), _row_spec(GH1),
                  _full_spec(GH1, 2 * GH2)],
        out_specs=_row_spec(2 * GH2),
        out_shape=jax.ShapeDtypeStruct((N, 2 * GH2), f32),
    )(p1[0], p1[1], mask2, gc23)

    p2 = spmm(y23, rows2d, cols2d, vals1d)

    s23, z, de_feat, q_pad = pl.pallas_call(
        _tail_body,
        grid=(GRID,),
        in_specs=[_row_spec(2 * GH2), _row_spec(2 * GH2), _row_spec(FH2),
                  _full_spec(FH2 + GH2, D), _full_spec(1, D),
                  _full_spec(1, D), _full_spec(1, D),
                  _full_spec(FH2 + GH2, 128), _full_spec(1, 128)],
        out_specs=[_row_spec(2 * GH2), _row_spec(FH2 + GH2), _row_spec(D),
                   _row_spec(128)],
        out_shape=[jax.ShapeDtypeStruct((N, 2 * GH2), f32),
                   jax.ShapeDtypeStruct((N, FH2 + GH2), f32),
                   jax.ShapeDtypeStruct((N, D), f32),
                   jax.ShapeDtypeStruct((N, 128), f32)],
    )(p2[0], p2[1], feat_x, dec_W, decb, decg, decbe, ct, c2)

    mu = s23[:, :GH2]
    logvar = s23[:, GH2:]
    q = q_pad[:, :KC]
    return (z, mu, logvar, de_feat, q, feat_x, mu)
